# Initial kernel scaffold; baseline (speedup 1.0000x reference)
import jax
import jax.numpy as jnp
from jax.experimental import pallas as pl


def kernel(image, boxes, box_indices):
    raise NotImplementedError("write your pallas kernel here")


# speedup vs baseline: 4.5183x; 4.5183x over previous
import jax
import jax.numpy as jnp
from jax.experimental import pallas as pl
from jax.experimental.pallas import tpu as pltpu

_RH, _RW = 64, 64


def _body(idx_ref, top_ref, bot_ref, wt_ref, wb_ref,
          top_blk, bot_blk, wxt_blk, out_blk):
    b = pl.program_id(0)
    ry = pl.program_id(1)
    wt = wt_ref[b, ry]
    wb = wb_ref[b, ry]
    row = top_blk[0, :, 0, 0, :] * wt + bot_blk[0, :, 0, 0, :] * wb
    out_blk[0, :, ry, :] = jnp.dot(row, wxt_blk[0],
                                   preferred_element_type=jnp.float32)


def kernel(image, boxes, box_indices):
    N, C, H, W = image.shape
    B = boxes.shape[0]

    y1 = boxes[:, 0]
    x1 = boxes[:, 1]
    y2 = boxes[:, 2]
    x2 = boxes[:, 3]
    hs = (y2 - y1) * (H - 1) / float(_RH - 1)
    ws = (x2 - x1) * (W - 1) / float(_RW - 1)
    ty = jnp.arange(_RH, dtype=jnp.float32)
    tx = jnp.arange(_RW, dtype=jnp.float32)
    in_y = y1[:, None] * (H - 1) + ty[None, :] * hs[:, None]
    in_x = x1[:, None] * (W - 1) + tx[None, :] * ws[:, None]
    yvalid = ((in_y >= 0) & (in_y <= H - 1)).astype(jnp.float32)
    xvalid = ((in_x >= 0) & (in_x <= W - 1)).astype(jnp.float32)
    in_y = jnp.where(yvalid > 0, in_y, 0.0)
    in_x = jnp.where(xvalid > 0, in_x, 0.0)
    top_y = jnp.floor(in_y)
    bot_y = jnp.ceil(in_y)
    left_x = jnp.floor(in_x)
    right_x = jnp.ceil(in_x)
    y_l = in_y - top_y
    x_l = in_x - left_x
    w_top = (1.0 - y_l) * yvalid
    w_bot = y_l * yvalid

    cols = jnp.arange(W, dtype=jnp.float32)[None, :, None]
    wxt = ((cols == left_x[:, None, :]) * (1.0 - x_l)[:, None, :]
           + (cols == right_x[:, None, :]) * x_l[:, None, :])
    wxt = (wxt * xvalid[:, None, :]).astype(jnp.float32)

    top_i = top_y.astype(jnp.int32)
    bot_i = bot_y.astype(jnp.int32)

    image5 = image.reshape(N, C, H, 1, W)

    grid_spec = pltpu.PrefetchScalarGridSpec(
        num_scalar_prefetch=5,
        grid=(B, _RH),
        in_specs=[
            pl.BlockSpec(
                (1, C, 1, 1, W),
                lambda b, ry, idx, top, bot, wt, wb: (idx[b], 0, top[b, ry], 0, 0)),
            pl.BlockSpec(
                (1, C, 1, 1, W),
                lambda b, ry, idx, top, bot, wt, wb: (idx[b], 0, bot[b, ry], 0, 0)),
            pl.BlockSpec(
                (1, W, _RW),
                lambda b, ry, idx, top, bot, wt, wb: (b, 0, 0)),
        ],
        out_specs=pl.BlockSpec(
            (1, C, _RH, _RW),
            lambda b, ry, idx, top, bot, wt, wb: (b, 0, 0, 0)),
    )

    return pl.pallas_call(
        _body,
        grid_spec=grid_spec,
        out_shape=jax.ShapeDtypeStruct((B, C, _RH, _RW), jnp.float32),
    )(box_indices.astype(jnp.int32), top_i, bot_i, w_top, w_bot,
      image5, image5, wxt)


# speedup vs baseline: 8.5296x; 1.8878x over previous
import jax
import jax.numpy as jnp
from jax.experimental import pallas as pl
from jax.experimental.pallas import tpu as pltpu

_RH, _RW = 64, 64
_RPS = 4


def _body(idx_ref, top_ref, bot_ref, wt_ref, wb_ref,
          *refs):
    row_blks = refs[:2 * _RPS]
    wxt_blk = refs[2 * _RPS]
    out_blk = refs[2 * _RPS + 1]
    b = pl.program_id(0)
    chunk = pl.program_id(1)
    wxt = wxt_blk[0]
    for i in range(_RPS):
        ry = chunk * _RPS + i
        wt = wt_ref[b, ry]
        wb = wb_ref[b, ry]
        top = row_blks[2 * i][0, :, 0, 0, :]
        bot = row_blks[2 * i + 1][0, :, 0, 0, :]
        row = top * wt + bot * wb
        out_blk[0, :, ry, :] = jnp.dot(row, wxt,
                                       preferred_element_type=jnp.float32)


def kernel(image, boxes, box_indices):
    N, C, H, W = image.shape
    B = boxes.shape[0]

    y1 = boxes[:, 0]
    x1 = boxes[:, 1]
    y2 = boxes[:, 2]
    x2 = boxes[:, 3]
    hs = (y2 - y1) * (H - 1) / float(_RH - 1)
    ws = (x2 - x1) * (W - 1) / float(_RW - 1)
    ty = jnp.arange(_RH, dtype=jnp.float32)
    tx = jnp.arange(_RW, dtype=jnp.float32)
    in_y = y1[:, None] * (H - 1) + ty[None, :] * hs[:, None]
    in_x = x1[:, None] * (W - 1) + tx[None, :] * ws[:, None]
    yvalid = ((in_y >= 0) & (in_y <= H - 1)).astype(jnp.float32)
    xvalid = ((in_x >= 0) & (in_x <= W - 1)).astype(jnp.float32)
    in_y = jnp.where(yvalid > 0, in_y, 0.0)
    in_x = jnp.where(xvalid > 0, in_x, 0.0)
    top_y = jnp.floor(in_y)
    bot_y = jnp.ceil(in_y)
    left_x = jnp.floor(in_x)
    right_x = jnp.ceil(in_x)
    y_l = in_y - top_y
    x_l = in_x - left_x
    w_top = (1.0 - y_l) * yvalid
    w_bot = y_l * yvalid

    cols = jnp.arange(W, dtype=jnp.float32)[None, :, None]
    wxt = ((cols == left_x[:, None, :]) * (1.0 - x_l)[:, None, :]
           + (cols == right_x[:, None, :]) * x_l[:, None, :])
    wxt = (wxt * xvalid[:, None, :]).astype(jnp.float32)

    top_i = top_y.astype(jnp.int32)
    bot_i = bot_y.astype(jnp.int32)

    image5 = image.reshape(N, C, H, 1, W)

    row_specs = []
    for i in range(_RPS):
        def _top_map(b, ch, idx, top, bot, wt, wb, _i=i):
            return (idx[b], 0, top[b, ch * _RPS + _i], 0, 0)

        def _bot_map(b, ch, idx, top, bot, wt, wb, _i=i):
            return (idx[b], 0, bot[b, ch * _RPS + _i], 0, 0)

        row_specs.append(pl.BlockSpec((1, C, 1, 1, W), _top_map))
        row_specs.append(pl.BlockSpec((1, C, 1, 1, W), _bot_map))

    grid_spec = pltpu.PrefetchScalarGridSpec(
        num_scalar_prefetch=5,
        grid=(B, _RH // _RPS),
        in_specs=row_specs + [
            pl.BlockSpec(
                (1, W, _RW),
                lambda b, ch, idx, top, bot, wt, wb: (b, 0, 0)),
        ],
        out_specs=pl.BlockSpec(
            (1, C, _RH, _RW),
            lambda b, ch, idx, top, bot, wt, wb: (b, 0, 0, 0)),
    )

    return pl.pallas_call(
        _body,
        grid_spec=grid_spec,
        out_shape=jax.ShapeDtypeStruct((B, C, _RH, _RW), jnp.float32),
    )(box_indices.astype(jnp.int32), top_i, bot_i, w_top, w_bot,
      *([image5] * (2 * _RPS)), wxt)


# speedup vs baseline: 10.1602x; 1.1912x over previous
import jax
import jax.numpy as jnp
from jax.experimental import pallas as pl
from jax.experimental.pallas import tpu as pltpu

_RH, _RW = 64, 64
_RPS = 8


def _body(idx_ref, top_ref, bot_ref, wt_ref, wb_ref,
          *refs):
    row_blks = refs[:2 * _RPS]
    wxt_blk = refs[2 * _RPS]
    out_blk = refs[2 * _RPS + 1]
    b = pl.program_id(0)
    chunk = pl.program_id(1)
    wxt = wxt_blk[0]
    for i in range(_RPS):
        ry = chunk * _RPS + i
        wt = wt_ref[b, ry]
        wb = wb_ref[b, ry]
        top = row_blks[2 * i][0, :, 0, 0, :]
        bot = row_blks[2 * i + 1][0, :, 0, 0, :]
        row = top * wt + bot * wb
        out_blk[0, :, ry, :] = jnp.dot(row, wxt,
                                       preferred_element_type=jnp.float32)


def kernel(image, boxes, box_indices):
    N, C, H, W = image.shape
    B = boxes.shape[0]

    y1 = boxes[:, 0]
    x1 = boxes[:, 1]
    y2 = boxes[:, 2]
    x2 = boxes[:, 3]
    hs = (y2 - y1) * (H - 1) / float(_RH - 1)
    ws = (x2 - x1) * (W - 1) / float(_RW - 1)
    ty = jnp.arange(_RH, dtype=jnp.float32)
    tx = jnp.arange(_RW, dtype=jnp.float32)
    in_y = y1[:, None] * (H - 1) + ty[None, :] * hs[:, None]
    in_x = x1[:, None] * (W - 1) + tx[None, :] * ws[:, None]
    yvalid = ((in_y >= 0) & (in_y <= H - 1)).astype(jnp.float32)
    xvalid = ((in_x >= 0) & (in_x <= W - 1)).astype(jnp.float32)
    in_y = jnp.where(yvalid > 0, in_y, 0.0)
    in_x = jnp.where(xvalid > 0, in_x, 0.0)
    top_y = jnp.floor(in_y)
    bot_y = jnp.ceil(in_y)
    left_x = jnp.floor(in_x)
    right_x = jnp.ceil(in_x)
    y_l = in_y - top_y
    x_l = in_x - left_x
    w_top = (1.0 - y_l) * yvalid
    w_bot = y_l * yvalid

    cols = jnp.arange(W, dtype=jnp.float32)[None, :, None]
    wxt = ((cols == left_x[:, None, :]) * (1.0 - x_l)[:, None, :]
           + (cols == right_x[:, None, :]) * x_l[:, None, :])
    wxt = (wxt * xvalid[:, None, :]).astype(jnp.float32)

    top_i = top_y.astype(jnp.int32)
    bot_i = bot_y.astype(jnp.int32)

    image5 = image.reshape(N, C, H, 1, W)

    row_specs = []
    for i in range(_RPS):
        def _top_map(b, ch, idx, top, bot, wt, wb, _i=i):
            return (idx[b], 0, top[b, ch * _RPS + _i], 0, 0)

        def _bot_map(b, ch, idx, top, bot, wt, wb, _i=i):
            return (idx[b], 0, bot[b, ch * _RPS + _i], 0, 0)

        row_specs.append(pl.BlockSpec((1, C, 1, 1, W), _top_map))
        row_specs.append(pl.BlockSpec((1, C, 1, 1, W), _bot_map))

    grid_spec = pltpu.PrefetchScalarGridSpec(
        num_scalar_prefetch=5,
        grid=(B, _RH // _RPS),
        in_specs=row_specs + [
            pl.BlockSpec(
                (1, W, _RW),
                lambda b, ch, idx, top, bot, wt, wb: (b, 0, 0)),
        ],
        out_specs=pl.BlockSpec(
            (1, C, _RH, _RW),
            lambda b, ch, idx, top, bot, wt, wb: (b, 0, 0, 0)),
    )

    return pl.pallas_call(
        _body,
        grid_spec=grid_spec,
        out_shape=jax.ShapeDtypeStruct((B, C, _RH, _RW), jnp.float32),
    )(box_indices.astype(jnp.int32), top_i, bot_i, w_top, w_bot,
      *([image5] * (2 * _RPS)), wxt)
